# baseline (speedup 1.0000x reference)
import functools

import jax
import jax.numpy as jnp
from jax import lax
from jax.experimental import pallas as pl
from jax.experimental.pallas import tpu as pltpu
from jax.experimental.pallas import tpu_sc as plsc

TOPK = 32
_NC, _NS = 2, 16
_NW = _NC * _NS


def _tc_kernel(x_ref, c_ref, am_ref, *, inv_sqrt_d):
    xb = x_ref[0]
    c = c_ref[...]
    logits_t = jax.lax.dot_general(
        c.astype(jnp.bfloat16), xb.astype(jnp.bfloat16),
        (((1,), (1,)), ((), ())),
        preferred_element_type=jnp.float32,
    ) * inv_sqrt_d
    m = jnp.max(logits_t, axis=0, keepdims=True)
    e = jnp.exp(logits_t - m)
    probs_t = e / jnp.sum(e, axis=0, keepdims=True)
    am_ref[...] = probs_t.T


def _merge_sorted16(av, ai, bv, bi):
    rbv = lax.rev(bv, (0,))
    rbi = lax.rev(bi, (0,))
    m = av >= rbv
    hv = jnp.where(m, av, rbv)
    hi_ = jnp.where(m, ai, rbi)
    lv = jnp.where(m, rbv, av)
    li_ = jnp.where(m, rbi, ai)
    hv, hi_ = plsc.sort_key_val(hv, hi_, descending=True)
    lv, li_ = plsc.sort_key_val(lv, li_, descending=True)
    return hv, hi_, lv, li_


def _merge_top32(a, b):
    ahv, ahi, alv, ali = a
    bhv, bhi, blv, bli = b
    rblv = lax.rev(blv, (0,))
    rbli = lax.rev(bli, (0,))
    m1 = ahv >= rblv
    t1v = jnp.where(m1, ahv, rblv)
    t1i = jnp.where(m1, ahi, rbli)
    rbhv = lax.rev(bhv, (0,))
    rbhi = lax.rev(bhi, (0,))
    m2 = alv >= rbhv
    t2v = jnp.where(m2, alv, rbhv)
    t2i = jnp.where(m2, ali, rbhi)
    m = t1v >= t2v
    uv = jnp.where(m, t1v, t2v)
    ui = jnp.where(m, t1i, t2i)
    lv = jnp.where(m, t2v, t1v)
    li_ = jnp.where(m, t2i, t1i)
    uv, ui = plsc.sort_key_val(uv, ui, descending=True)
    lv, li_ = plsc.sort_key_val(lv, li_, descending=True)
    return uv, ui, lv, li_


def _sc_topk_body(probs_hbm, sims_hbm, idx_hbm, buf, sims_v, idx_v, sem_in):
    wid = lax.axis_index("s") * _NC + lax.axis_index("c")
    n_s, n_b, kdim = probs_hbm.shape
    per_b = _NW // n_b
    rows = n_s // per_b
    b = wid // per_b
    s0 = (wid % per_b) * rows

    pltpu.async_copy(probs_hbm.at[s0, b], buf.at[0], sem_in)

    def item(i, _):
        pb = lax.rem(i, 2)
        pltpu.make_async_copy(probs_hbm.at[s0, b], buf.at[pb], sem_in).wait()

        @pl.when(i + 1 < rows)
        def _start_next():
            pltpu.async_copy(probs_hbm.at[s0 + i + 1, b],
                             buf.at[lax.rem(i + 1, 2)], sem_in)

        lists = []
        for j in range(kdim // 32):
            av = buf[pb, pl.ds(32 * j, 16)]
            bv = buf[pb, pl.ds(32 * j + 16, 16)]
            ia = lax.iota(jnp.int32, 16) + (32 * j)
            ib = lax.iota(jnp.int32, 16) + (32 * j + 16)
            av, ia = plsc.sort_key_val(av, ia, descending=True)
            bv, ib = plsc.sort_key_val(bv, ib, descending=True)
            lists.append(_merge_sorted16(av, ia, bv, ib))
        while len(lists) > 1:
            lists = [_merge_top32(a, b2)
                     for a, b2 in zip(lists[0::2], lists[1::2])]
        hv, hi_, lv, li_ = lists[0]

        ii = lax.rem(i, 16)
        sims_v[ii, pl.ds(0, 16)] = hv
        sims_v[ii, pl.ds(16, 16)] = lv
        idx_v[ii, pl.ds(0, 16)] = hi_
        idx_v[ii, pl.ds(16, 16)] = li_

        @pl.when(lax.rem(i, 16) == 15)
        def _flush():
            r0 = pl.multiple_of(b * n_s + s0 + i - 15, 16)
            pltpu.sync_copy(sims_v, sims_hbm.at[pl.ds(r0, 16)])
            pltpu.sync_copy(idx_v, idx_hbm.at[pl.ds(r0, 16)])

        return 0

    lax.fori_loop(0, rows, item, 0)


@jax.jit
def kernel(x, centroids):
    B, S, D = x.shape
    K = centroids.shape[0]
    S_blk = 512
    grid = (B, S // S_blk)
    body = functools.partial(_tc_kernel, inv_sqrt_d=1.0 / (D ** 0.5))
    amatrix_r = pl.pallas_call(
        body,
        grid=grid,
        in_specs=[
            pl.BlockSpec((1, S_blk, D), lambda b, s: (b, s, 0)),
            pl.BlockSpec((K, D), lambda b, s: (0, 0)),
        ],
        out_specs=pl.BlockSpec((S_blk, K), lambda b, s: (s, b)),
        out_shape=jax.ShapeDtypeStruct((S, B * K), jnp.float32),
    )(x, centroids)

    mesh = plsc.VectorSubcoreMesh(core_axis_name="c", subcore_axis_name="s",
                                  num_cores=_NC, num_subcores=_NS)
    sc_topk = functools.partial(
        pl.kernel, mesh=mesh,
        out_type=[jax.ShapeDtypeStruct((B * S, TOPK), jnp.float32),
                  jax.ShapeDtypeStruct((B * S, TOPK), jnp.int32)],
        scratch_types=[pltpu.VMEM((2, K), jnp.float32),
                       pltpu.VMEM((16, TOPK), jnp.float32),
                       pltpu.VMEM((16, TOPK), jnp.int32),
                       pltpu.SemaphoreType.DMA],
        compiler_params=pltpu.CompilerParams(needs_layout_passes=False),
    )(_sc_topk_body)
    sims, indices = sc_topk(amatrix_r.reshape(S, B, K))
    return (sims.reshape(B, S, TOPK), indices.reshape(B, S, TOPK),
            amatrix_r)


# speedup vs baseline: 1.3313x; 1.3313x over previous
import functools

import jax
import jax.numpy as jnp
from jax.experimental import pallas as pl

TOPK = 32


def _fused_kernel(x_ref, c_ref, sims_ref, idx_ref, am_ref, *, inv_sqrt_d):
    xb = x_ref[0]
    c = c_ref[...]
    logits_t = jax.lax.dot_general(
        c.astype(jnp.bfloat16), xb.astype(jnp.bfloat16),
        (((1,), (1,)), ((), ())),
        preferred_element_type=jnp.float32,
    ) * inv_sqrt_d
    m = jnp.max(logits_t, axis=0, keepdims=True)
    e = jnp.exp(logits_t - m)
    probs_t = e / jnp.sum(e, axis=0, keepdims=True)
    am_ref[...] = probs_t.T

    iota = jax.lax.broadcasted_iota(jnp.int32, probs_t.shape, 0)
    vals = probs_t
    sims_rows = []
    idx_rows = []
    for _ in range(TOPK):
        v, ix = vals, iota
        while v.shape[0] > 1:
            h = v.shape[0] // 2
            a_v, b_v = v[:h], v[h:]
            take = a_v >= b_v
            v = jnp.maximum(a_v, b_v)
            ix = jnp.where(take, ix[:h], ix[h:])
        sims_rows.append(v)
        idx_rows.append(ix)
        vals = jnp.where(iota == ix, -1.0, vals)

    for start in (0, 1, 0):
        for j in range(start, TOPK - 1, 2):
            tie = sims_rows[j] == sims_rows[j + 1]
            lo = jnp.minimum(idx_rows[j], idx_rows[j + 1])
            hi = jnp.maximum(idx_rows[j], idx_rows[j + 1])
            idx_rows[j] = jnp.where(tie, lo, idx_rows[j])
            idx_rows[j + 1] = jnp.where(tie, hi, idx_rows[j + 1])

    sims_ref[0] = jnp.concatenate(sims_rows, axis=0).T
    idx_ref[0] = jnp.concatenate(idx_rows, axis=0).T


@jax.jit
def kernel(x, centroids):
    B, S, D = x.shape
    K = centroids.shape[0]
    S_blk = 512
    grid = (B, S // S_blk)
    body = functools.partial(_fused_kernel, inv_sqrt_d=1.0 / (D ** 0.5))
    sims, indices, amatrix_r = pl.pallas_call(
        body,
        grid=grid,
        in_specs=[
            pl.BlockSpec((1, S_blk, D), lambda b, s: (b, s, 0)),
            pl.BlockSpec((K, D), lambda b, s: (0, 0)),
        ],
        out_specs=[
            pl.BlockSpec((1, S_blk, TOPK), lambda b, s: (b, s, 0)),
            pl.BlockSpec((1, S_blk, TOPK), lambda b, s: (b, s, 0)),
            pl.BlockSpec((S_blk, K), lambda b, s: (s, b)),
        ],
        out_shape=[
            jax.ShapeDtypeStruct((B, S, TOPK), jnp.float32),
            jax.ShapeDtypeStruct((B, S, TOPK), jnp.int32),
            jax.ShapeDtypeStruct((S, B * K), jnp.float32),
        ],
    )(x, centroids)
    return sims, indices, amatrix_r
